# baseline (speedup 1.0000x reference)
import functools

import jax
import jax.numpy as jnp
from jax import lax
from jax.experimental import pallas as pl
from jax.experimental.pallas import tpu as pltpu
from jax.experimental.pallas import tpu_sc as plsc

NUM_NODE = 10000
NUM_RELATION = 64
HIDDEN = 128
BATCH = 16384

NC, NS = 2, 16
NW = NC * NS
BPW = BATCH // NW
CH = 128
NCH = BPW // CH
GPC = CH // 16
HU = 4

_sc_mesh = plsc.VectorSubcoreMesh(core_axis_name="c", subcore_axis_name="s")


@functools.partial(
    pl.kernel,
    mesh=_sc_mesh,
    compiler_params=pltpu.CompilerParams(needs_layout_passes=False),
    out_type=(
        jax.ShapeDtypeStruct((BATCH, 2), jnp.float32),
        jax.ShapeDtypeStruct((BATCH // 16, 16), jnp.float32),
    ),
    scratch_types=[
        pltpu.VMEM((2, NCH, CH), jnp.int32),
        pltpu.VMEM((BPW // 16, 16), jnp.int32),
        pltpu.VMEM((NUM_RELATION, HIDDEN), jnp.float32),
        pltpu.VMEM((2, CH, HIDDEN), jnp.float32),
        pltpu.VMEM((2, CH, HIDDEN), jnp.float32),
        pltpu.VMEM((CH, 2), jnp.float32),
        pltpu.VMEM((GPC, 16), jnp.float32),
        pltpu.SemaphoreType.DMA,
        pltpu.SemaphoreType.DMA,
    ],
)
def _sc_fused(idx_hbm, r_hbm, wx_hbm, wr_hbm, out_lg, out_p,
              idx_v, r_v, wr_v, xb, yb, lg_v, p_v, s0, s1):
    wid = lax.axis_index("s") * NC + lax.axis_index("c")
    base = wid * BPW

    pltpu.sync_copy(idx_hbm.at[wid], idx_v)
    pltpu.sync_copy(r_hbm.at[wid], r_v)
    pltpu.sync_copy(wr_hbm, wr_v)

    def _reg_row(k, carry):
        for j in range(HIDDEN // 16):
            v = wr_v[k, pl.ds(j * 16, 16)]
            v = jnp.clip(v, -6.0, 6.0)
            s = 1.0 / (1.0 + jnp.exp(-v))
            wr_v[k, pl.ds(j * 16, 16)] = s * (1.0 - s)
        return carry

    lax.fori_loop(0, NUM_RELATION, _reg_row, 0)

    sems = (s0, s1)
    bufs = ((xb.at[0], yb.at[0]), (xb.at[1], yb.at[1]))
    copies = {}

    def _fire(c):
        b = c % 2
        cpx = pltpu.async_copy(wx_hbm.at[idx_v.at[0, c]], bufs[b][0], sems[b])
        cpy = pltpu.async_copy(wx_hbm.at[idx_v.at[1, c]], bufs[b][1], sems[b])
        copies[c] = (cpx, cpy)

    iota16 = lax.broadcasted_iota(jnp.int32, (16,), 0)
    zeros16 = jnp.zeros((16,), jnp.int32)
    ones16 = jnp.full((16,), 1, jnp.int32)

    _fire(0)
    for c in range(NCH):
        if c + 1 < NCH:
            _fire(c + 1)
        cpx, cpy = copies[c]
        cpx.wait()
        cpy.wait()
        xr, yr = bufs[c % 2]
        for g in range(GPC):
            rvec = r_v[c * GPC + g]
            rows = g * 16 + iota16

            def _hbody(k, acc, xr=xr, yr=yr, rvec=rvec, rows=rows):
                for u in range(HU):
                    col = jnp.full((16,), k * HU + u, jnp.int32)
                    vx = plsc.load_gather(xr, [rows, col])
                    vy = plsc.load_gather(yr, [rows, col])
                    vre = plsc.load_gather(wr_v, [rvec, col])
                    acc = acc + vx * vy * vre
                return acc

            acc = lax.fori_loop(0, HIDDEN // HU, _hbody,
                                jnp.zeros((16,), jnp.float32))
            p = 1.0 / (1.0 + jnp.exp(-acc))
            plsc.store_scatter(lg_v, [rows, zeros16], p)
            plsc.store_scatter(lg_v, [rows, ones16], 1.0 - p)
            p_v[g] = p
        pltpu.sync_copy(lg_v, out_lg.at[pl.ds(base + c * CH, CH)])
        pltpu.sync_copy(p_v, out_p.at[pl.ds(wid * (BPW // 16) + c * GPC, GPC)])


def _loss_body(p_ref, l_ref, loss_ref):
    p = p_ref[...]
    lse = jnp.log(jnp.exp(p) + jnp.exp(1.0 - p))
    chosen = jnp.where(l_ref[...] == 0, p, 1.0 - p)
    loss_ref[...] = (jnp.sum(lse - chosen) * (1.0 / BATCH)).reshape(1, 1)


_tc_loss = pl.pallas_call(
    _loss_body,
    out_shape=jax.ShapeDtypeStruct((1, 1), jnp.float32),
)


def kernel(x, y, r, l, Wx, Wr):
    xi = x.astype(jnp.int32).reshape(NW, NCH, CH)
    yi = y.astype(jnp.int32).reshape(NW, NCH, CH)
    idx = jnp.stack([xi, yi], axis=1)
    ri = r.astype(jnp.int32).reshape(NW, BPW // 16, 16)
    logits, preds = _sc_fused(idx, ri, Wx, Wr)
    loss = _tc_loss(preds.reshape(HIDDEN, HIDDEN),
                    l.astype(jnp.int32).reshape(HIDDEN, HIDDEN))
    return logits, loss.reshape(())


# speedup vs baseline: 2.5817x; 2.5817x over previous
import functools

import jax
import jax.numpy as jnp
from jax import lax
from jax.experimental import pallas as pl
from jax.experimental.pallas import tpu as pltpu
from jax.experimental.pallas import tpu_sc as plsc

NUM_NODE = 10000
NUM_RELATION = 64
HIDDEN = 128
BATCH = 16384

NC, NS = 2, 16
NW = NC * NS
BPW = BATCH // NW
CH = 128
NCH = BPW // CH
GPC = CH // 16
HU = 4

_sc_mesh = plsc.VectorSubcoreMesh(core_axis_name="c", subcore_axis_name="s")


@functools.partial(
    pl.kernel,
    mesh=_sc_mesh,
    compiler_params=pltpu.CompilerParams(needs_layout_passes=False),
    out_type=(
        jax.ShapeDtypeStruct((BATCH, 2), jnp.float32),
        jax.ShapeDtypeStruct((BATCH // 16, 16), jnp.float32),
    ),
    scratch_types=[
        pltpu.VMEM((2, NCH, CH), jnp.int32),
        pltpu.VMEM((BPW // 16, 16), jnp.int32),
        pltpu.VMEM((NUM_RELATION, HIDDEN), jnp.float32),
        pltpu.VMEM((2, CH, HIDDEN), jnp.float32),
        pltpu.VMEM((2, CH, HIDDEN), jnp.float32),
        pltpu.VMEM((CH, 2), jnp.float32),
        pltpu.VMEM((GPC, 16), jnp.float32),
        pltpu.SemaphoreType.DMA,
        pltpu.SemaphoreType.DMA,
    ],
)
def _sc_fused(idx_hbm, r_hbm, wx_hbm, wr_hbm, out_lg, out_p,
              idx_v, r_v, wr_v, xb, yb, lg_v, p_v, s0, s1):
    wid = lax.axis_index("s") * NC + lax.axis_index("c")
    base = wid * BPW

    pltpu.sync_copy(idx_hbm.at[wid], idx_v)
    pltpu.sync_copy(r_hbm.at[wid], r_v)
    pltpu.sync_copy(wr_hbm, wr_v)

    def _reg_row(k, carry):
        for j in range(HIDDEN // 16):
            v = wr_v[k, pl.ds(j * 16, 16)]
            v = jnp.clip(v, -6.0, 6.0)
            s = 1.0 / (1.0 + jnp.exp(-v))
            wr_v[k, pl.ds(j * 16, 16)] = s * (1.0 - s)
        return carry

    lax.fori_loop(0, NUM_RELATION, _reg_row, 0)

    sems = (s0, s1)
    bufs = ((xb.at[0], yb.at[0]), (xb.at[1], yb.at[1]))
    copies = {}

    def _fire(c):
        b = c % 2
        cpx = pltpu.async_copy(wx_hbm.at[idx_v.at[0, c]], bufs[b][0], sems[b])
        cpy = pltpu.async_copy(wx_hbm.at[idx_v.at[1, c]], bufs[b][1], sems[b])
        copies[c] = (cpx, cpy)

    iota16 = lax.broadcasted_iota(jnp.int32, (16,), 0)
    zeros16 = jnp.zeros((16,), jnp.int32)
    ones16 = jnp.full((16,), 1, jnp.int32)

    _fire(0)
    for c in range(NCH):
        if c + 1 < NCH:
            _fire(c + 1)
        cpx, cpy = copies[c]
        cpx.wait()
        cpy.wait()
        xr, yr = bufs[c % 2]
        for g in range(GPC):
            rvec = r_v[c * GPC + g]
            rows = g * 16 + iota16

            def _hbody(k, acc, xr=xr, yr=yr, rvec=rvec, rows=rows):
                for u in range(HU):
                    col = (iota16 + (k * HU + u)) & (HIDDEN - 1)
                    vx = plsc.load_gather(xr, [rows, col])
                    vy = plsc.load_gather(yr, [rows, col])
                    vre = plsc.load_gather(wr_v, [rvec, col])
                    acc = acc + vx * vy * vre
                return acc

            acc = lax.fori_loop(0, HIDDEN // HU, _hbody,
                                jnp.zeros((16,), jnp.float32))
            p = 1.0 / (1.0 + jnp.exp(-acc))
            plsc.store_scatter(lg_v, [rows, zeros16], p)
            plsc.store_scatter(lg_v, [rows, ones16], 1.0 - p)
            p_v[g] = p
        pltpu.sync_copy(lg_v, out_lg.at[pl.ds(base + c * CH, CH)])
        pltpu.sync_copy(p_v, out_p.at[pl.ds(wid * (BPW // 16) + c * GPC, GPC)])


def _loss_body(p_ref, l_ref, loss_ref):
    p = p_ref[...]
    lse = jnp.log(jnp.exp(p) + jnp.exp(1.0 - p))
    chosen = jnp.where(l_ref[...] == 0, p, 1.0 - p)
    loss_ref[...] = (jnp.sum(lse - chosen) * (1.0 / BATCH)).reshape(1, 1)


_tc_loss = pl.pallas_call(
    _loss_body,
    out_shape=jax.ShapeDtypeStruct((1, 1), jnp.float32),
)


def kernel(x, y, r, l, Wx, Wr):
    xi = x.astype(jnp.int32).reshape(NW, NCH, CH)
    yi = y.astype(jnp.int32).reshape(NW, NCH, CH)
    idx = jnp.stack([xi, yi], axis=1)
    ri = r.astype(jnp.int32).reshape(NW, BPW // 16, 16)
    logits, preds = _sc_fused(idx, ri, Wx, Wr)
    loss = _tc_loss(preds.reshape(HIDDEN, HIDDEN),
                    l.astype(jnp.int32).reshape(HIDDEN, HIDDEN))
    return logits, loss.reshape(())


# speedup vs baseline: 2.6727x; 1.0353x over previous
import functools

import jax
import jax.numpy as jnp
from jax import lax
from jax.experimental import pallas as pl
from jax.experimental.pallas import tpu as pltpu
from jax.experimental.pallas import tpu_sc as plsc

NUM_NODE = 10000
NUM_RELATION = 64
HIDDEN = 128
BATCH = 16384

NC, NS = 2, 16
NW = NC * NS
BPW = BATCH // NW
CH = 128
NCH = BPW // CH
GPC = CH // 16
HU = 4
NBUF = 2

_sc_mesh = plsc.VectorSubcoreMesh(core_axis_name="c", subcore_axis_name="s")


@functools.partial(
    pl.kernel,
    mesh=_sc_mesh,
    compiler_params=pltpu.CompilerParams(needs_layout_passes=False),
    out_type=(
        jax.ShapeDtypeStruct((BATCH, 2), jnp.float32),
        jax.ShapeDtypeStruct((BATCH // 16, 16), jnp.float32),
        jax.ShapeDtypeStruct((BATCH // 16, 16), jnp.int32),
    ),
    scratch_types=[
        pltpu.VMEM((2, NCH, CH), jnp.int32),
        pltpu.VMEM((BPW // 16, 16), jnp.int32),
        pltpu.VMEM((BPW // 16, 16), jnp.int32),
        pltpu.VMEM((NUM_RELATION, HIDDEN), jnp.float32),
        pltpu.VMEM((NBUF, CH, HIDDEN), jnp.float32),
        pltpu.VMEM((NBUF, CH, HIDDEN), jnp.float32),
        pltpu.VMEM((CH, 2), jnp.float32),
        pltpu.VMEM((GPC, 16), jnp.float32),
        pltpu.SemaphoreType.DMA,
        pltpu.SemaphoreType.DMA,
    ],
)
def _sc_fused(x_hbm, y_hbm, r_hbm, l_hbm, wx_hbm, wr_hbm,
              out_lg, out_p, out_l,
              idx_v, r_v, l_v, wr_v, xb, yb, lg_v, p_v, s0, s1):
    wid = lax.axis_index("s") * NC + lax.axis_index("c")
    base = wid * BPW

    pltpu.sync_copy(x_hbm.at[wid], idx_v.at[0])
    pltpu.sync_copy(y_hbm.at[wid], idx_v.at[1])
    pltpu.sync_copy(r_hbm.at[wid], r_v)

    sems = (s0, s1)
    copies = {}

    def _fire(c):
        b = c % NBUF
        cpx = pltpu.async_copy(wx_hbm.at[idx_v.at[0, c]], xb.at[b], sems[b])
        cpy = pltpu.async_copy(wx_hbm.at[idx_v.at[1, c]], yb.at[b], sems[b])
        copies[c] = (cpx, cpy)

    for c in range(NBUF - 1):
        _fire(c)

    pltpu.sync_copy(l_hbm.at[wid], l_v)
    pltpu.sync_copy(l_v, out_l.at[pl.ds(wid * (BPW // 16), BPW // 16)])

    pltpu.sync_copy(wr_hbm, wr_v)

    def _reg_row(k, carry):
        for j in range(HIDDEN // 16):
            v = wr_v[k, pl.ds(j * 16, 16)]
            v = jnp.clip(v, -6.0, 6.0)
            s = 1.0 / (1.0 + jnp.exp(-v))
            wr_v[k, pl.ds(j * 16, 16)] = s * (1.0 - s)
        return carry

    lax.fori_loop(0, NUM_RELATION, _reg_row, 0)

    iota16 = lax.broadcasted_iota(jnp.int32, (16,), 0)
    zeros16 = jnp.zeros((16,), jnp.int32)
    ones16 = jnp.full((16,), 1, jnp.int32)

    for c in range(NCH):
        if c + NBUF - 1 < NCH:
            _fire(c + NBUF - 1)
        cpx, cpy = copies[c]
        cpx.wait()
        cpy.wait()
        b = c % NBUF
        xr, yr = xb.at[b], yb.at[b]
        for g in range(GPC):
            rvec = r_v[c * GPC + g]
            rows = g * 16 + iota16

            def _hbody(k, acc, xr=xr, yr=yr, rvec=rvec, rows=rows):
                for u in range(HU):
                    col = (iota16 + (k * HU + u)) & (HIDDEN - 1)
                    vx = plsc.load_gather(xr, [rows, col])
                    vy = plsc.load_gather(yr, [rows, col])
                    vre = plsc.load_gather(wr_v, [rvec, col])
                    acc = acc + vx * vy * vre
                return acc

            acc = lax.fori_loop(0, HIDDEN // HU, _hbody,
                                jnp.zeros((16,), jnp.float32))
            p = 1.0 / (1.0 + jnp.exp(-acc))
            plsc.store_scatter(lg_v, [rows, zeros16], p)
            plsc.store_scatter(lg_v, [rows, ones16], 1.0 - p)
            p_v[g] = p
        pltpu.sync_copy(lg_v, out_lg.at[pl.ds(base + c * CH, CH)])
        pltpu.sync_copy(p_v, out_p.at[pl.ds(wid * (BPW // 16) + c * GPC, GPC)])


def _loss_body(p_ref, l_ref, loss_ref):
    p = p_ref[...]
    lse = jnp.log(jnp.exp(p) + jnp.exp(1.0 - p))
    chosen = jnp.where(l_ref[...] == 0, p, 1.0 - p)
    loss_ref[...] = (jnp.sum(lse - chosen) * (1.0 / BATCH)).reshape(1, 1)


_tc_loss = pl.pallas_call(
    _loss_body,
    out_shape=jax.ShapeDtypeStruct((1, 1), jnp.float32),
)


def kernel(x, y, r, l, Wx, Wr):
    xi = x.astype(jnp.int32).reshape(NW, NCH, CH)
    yi = y.astype(jnp.int32).reshape(NW, NCH, CH)
    ri = r.astype(jnp.int32).reshape(NW, BPW // 16, 16)
    li = l.astype(jnp.int32).reshape(NW, BPW // 16, 16)
    logits, preds, lsq = _sc_fused(xi, yi, ri, li, Wx, Wr)
    loss = _tc_loss(preds, lsq)
    return logits, loss.reshape(())


# speedup vs baseline: 2.8951x; 1.0832x over previous
import functools

import jax
import jax.numpy as jnp
from jax import lax
from jax.experimental import pallas as pl
from jax.experimental.pallas import tpu as pltpu
from jax.experimental.pallas import tpu_sc as plsc

NUM_NODE = 10000
NUM_RELATION = 64
HIDDEN = 128
BATCH = 16384

NC, NS = 2, 16
NW = NC * NS
BPW = BATCH // NW
CH = 128
NCH = BPW // CH
GPC = CH // 16
HU = 4
NBUF = 2

_sc_mesh = plsc.VectorSubcoreMesh(core_axis_name="c", subcore_axis_name="s")


@functools.partial(
    pl.kernel,
    mesh=_sc_mesh,
    compiler_params=pltpu.CompilerParams(needs_layout_passes=False),
    out_type=(
        jax.ShapeDtypeStruct((BATCH, 2), jnp.float32),
        jax.ShapeDtypeStruct((BATCH // 16, 16), jnp.float32),
        jax.ShapeDtypeStruct((BATCH // 16, 16), jnp.float32),
    ),
    scratch_types=[
        pltpu.VMEM((2, NCH * CH), jnp.int32),
        pltpu.VMEM((BPW,), jnp.int32),
        pltpu.VMEM((BPW,), jnp.int32),
        pltpu.VMEM((NUM_RELATION, HIDDEN), jnp.float32),
        pltpu.VMEM((NBUF, CH, HIDDEN), jnp.float32),
        pltpu.VMEM((NBUF, CH, HIDDEN), jnp.float32),
        pltpu.VMEM((2, CH, 2), jnp.float32),
        pltpu.VMEM((2, GPC, 16), jnp.float32),
        pltpu.VMEM((2, GPC, 16), jnp.float32),
        pltpu.SemaphoreType.DMA,
        pltpu.SemaphoreType.DMA,
        pltpu.SemaphoreType.DMA,
        pltpu.SemaphoreType.DMA,
        pltpu.SemaphoreType.DMA,
    ],
)
def _sc_fused(x_hbm, y_hbm, r_hbm, l_hbm, wx_hbm, wr_hbm,
              out_lg, out_p, out_ch,
              idx_v, r_v, l_v, wr_v, xb, yb, lg_v, p_v, ch_v,
              s0, s1, so0, so1, si):
    wid = lax.axis_index("s") * NC + lax.axis_index("c")
    base = wid * BPW
    obase = wid * (BPW // 16)

    st = [
        pltpu.async_copy(x_hbm.at[pl.ds(base, BPW)], idx_v.at[0], si),
        pltpu.async_copy(y_hbm.at[pl.ds(base, BPW)], idx_v.at[1], si),
        pltpu.async_copy(r_hbm.at[pl.ds(base, BPW)], r_v, si),
        pltpu.async_copy(l_hbm.at[pl.ds(base, BPW)], l_v, si),
        pltpu.async_copy(wr_hbm, wr_v, si),
    ]
    for cp in st[:2]:
        cp.wait()

    gsems = (s0, s1)
    gcopies = {}

    def _fire(c):
        b = c % NBUF
        cpx = pltpu.async_copy(
            wx_hbm.at[idx_v.at[0, pl.ds(c * CH, CH)]], xb.at[b], gsems[b])
        cpy = pltpu.async_copy(
            wx_hbm.at[idx_v.at[1, pl.ds(c * CH, CH)]], yb.at[b], gsems[b])
        gcopies[c] = (cpx, cpy)

    for c in range(NBUF - 1):
        _fire(c)
    for cp in st[2:]:
        cp.wait()

    def _reg_row(k, carry):
        for j in range(HIDDEN // 16):
            v = wr_v[k, pl.ds(j * 16, 16)]
            v = jnp.clip(v, -6.0, 6.0)
            s = 1.0 / (1.0 + jnp.exp(-v))
            wr_v[k, pl.ds(j * 16, 16)] = s * (1.0 - s)
        return carry

    lax.fori_loop(0, NUM_RELATION, _reg_row, 0)

    iota16 = lax.broadcasted_iota(jnp.int32, (16,), 0)
    zeros16 = jnp.zeros((16,), jnp.int32)
    ones16 = jnp.full((16,), 1, jnp.int32)
    osems = (so0, so1)
    ocopies = {}

    for c in range(NCH):
        if c + NBUF - 1 < NCH:
            _fire(c + NBUF - 1)
        cpx, cpy = gcopies[c]
        cpx.wait()
        cpy.wait()
        gb = c % NBUF
        xr, yr = xb.at[gb], yb.at[gb]
        ob = c % 2
        if c - 2 >= 0:
            for cp in ocopies[c - 2]:
                cp.wait()
        for g in range(GPC):
            rvec = r_v[pl.ds((c * GPC + g) * 16, 16)]
            lvec = l_v[pl.ds((c * GPC + g) * 16, 16)]
            rows = g * 16 + iota16

            def _hbody(k, acc, xr=xr, yr=yr, rvec=rvec, rows=rows):
                for u in range(HU):
                    col = (iota16 + (k * HU + u)) & (HIDDEN - 1)
                    vx = plsc.load_gather(xr, [rows, col])
                    vy = plsc.load_gather(yr, [rows, col])
                    vre = plsc.load_gather(wr_v, [rvec, col])
                    acc = acc + vx * vy * vre
                return acc

            acc = lax.fori_loop(0, HIDDEN // HU, _hbody,
                                jnp.zeros((16,), jnp.float32))
            p = 1.0 / (1.0 + jnp.exp(-acc))
            plsc.store_scatter(lg_v.at[ob], [rows, zeros16], p)
            plsc.store_scatter(lg_v.at[ob], [rows, ones16], 1.0 - p)
            p_v[ob, g] = p
            ch_v[ob, g] = jnp.where(lvec == 0, p, 1.0 - p)
        ocopies[c] = (
            pltpu.async_copy(lg_v.at[ob], out_lg.at[pl.ds(base + c * CH, CH)],
                             osems[ob]),
            pltpu.async_copy(p_v.at[ob], out_p.at[pl.ds(obase + c * GPC, GPC)],
                             osems[ob]),
            pltpu.async_copy(ch_v.at[ob], out_ch.at[pl.ds(obase + c * GPC, GPC)],
                             osems[ob]),
        )
    for c in (NCH - 2, NCH - 1):
        for cp in ocopies[c]:
            cp.wait()


def _loss_body(p_ref, ch_ref, loss_ref):
    p = p_ref[...]
    lse = jnp.log(jnp.exp(p) + jnp.exp(1.0 - p))
    loss_ref[...] = (jnp.sum(lse - ch_ref[...]) * (1.0 / BATCH)).reshape(1, 1)


_tc_loss = pl.pallas_call(
    _loss_body,
    out_shape=jax.ShapeDtypeStruct((1, 1), jnp.float32),
)


def kernel(x, y, r, l, Wx, Wr):
    logits, preds, chosen = _sc_fused(
        x.astype(jnp.int32), y.astype(jnp.int32),
        r.astype(jnp.int32), l.astype(jnp.int32), Wx, Wr)
    loss = _tc_loss(preds, chosen)
    return logits, loss.reshape(())


# speedup vs baseline: 2.9130x; 1.0062x over previous
import functools

import jax
import jax.numpy as jnp
from jax import lax
from jax.experimental import pallas as pl
from jax.experimental.pallas import tpu as pltpu
from jax.experimental.pallas import tpu_sc as plsc

NUM_NODE = 10000
NUM_RELATION = 64
HIDDEN = 128
BATCH = 16384

NC, NS = 2, 16
NW = NC * NS
BPW = BATCH // NW
CH = 64
NCH = BPW // CH
GPC = CH // 16
HU = 4
NBUF = 3

_sc_mesh = plsc.VectorSubcoreMesh(core_axis_name="c", subcore_axis_name="s")


@functools.partial(
    pl.kernel,
    mesh=_sc_mesh,
    compiler_params=pltpu.CompilerParams(needs_layout_passes=False),
    out_type=(
        jax.ShapeDtypeStruct((BATCH, 2), jnp.float32),
        jax.ShapeDtypeStruct((BATCH // 16, 16), jnp.float32),
        jax.ShapeDtypeStruct((BATCH // 16, 16), jnp.float32),
    ),
    scratch_types=[
        pltpu.VMEM((2, NCH * CH), jnp.int32),
        pltpu.VMEM((BPW,), jnp.int32),
        pltpu.VMEM((BPW,), jnp.int32),
        pltpu.VMEM((NUM_RELATION, HIDDEN), jnp.float32),
        pltpu.VMEM((NBUF, CH, HIDDEN), jnp.float32),
        pltpu.VMEM((NBUF, CH, HIDDEN), jnp.float32),
        pltpu.VMEM((2, CH, 2), jnp.float32),
        pltpu.VMEM((2, GPC, 16), jnp.float32),
        pltpu.VMEM((2, GPC, 16), jnp.float32),
        pltpu.SemaphoreType.DMA,
        pltpu.SemaphoreType.DMA,
        pltpu.SemaphoreType.DMA,
        pltpu.SemaphoreType.DMA,
        pltpu.SemaphoreType.DMA,
        pltpu.SemaphoreType.DMA,
    ],
)
def _sc_fused(x_hbm, y_hbm, r_hbm, l_hbm, wx_hbm, wr_hbm,
              out_lg, out_p, out_ch,
              idx_v, r_v, l_v, wr_v, xb, yb, lg_v, p_v, ch_v,
              s0, s1, s2, so0, so1, si):
    wid = lax.axis_index("s") * NC + lax.axis_index("c")
    base = wid * BPW
    obase = wid * (BPW // 16)

    st = [
        pltpu.async_copy(x_hbm.at[pl.ds(base, BPW)], idx_v.at[0], si),
        pltpu.async_copy(y_hbm.at[pl.ds(base, BPW)], idx_v.at[1], si),
        pltpu.async_copy(r_hbm.at[pl.ds(base, BPW)], r_v, si),
        pltpu.async_copy(l_hbm.at[pl.ds(base, BPW)], l_v, si),
        pltpu.async_copy(wr_hbm, wr_v, si),
    ]
    for cp in st[:2]:
        cp.wait()

    gsems = (s0, s1, s2)
    gcopies = {}

    def _fire(c):
        b = c % NBUF
        cpx = pltpu.async_copy(
            wx_hbm.at[idx_v.at[0, pl.ds(c * CH, CH)]], xb.at[b], gsems[b])
        cpy = pltpu.async_copy(
            wx_hbm.at[idx_v.at[1, pl.ds(c * CH, CH)]], yb.at[b], gsems[b])
        gcopies[c] = (cpx, cpy)

    for c in range(NBUF - 1):
        _fire(c)
    for cp in st[2:]:
        cp.wait()

    def _reg_row(k, carry):
        for j in range(HIDDEN // 16):
            v = wr_v[k, pl.ds(j * 16, 16)]
            v = jnp.clip(v, -6.0, 6.0)
            s = 1.0 / (1.0 + jnp.exp(-v))
            wr_v[k, pl.ds(j * 16, 16)] = s * (1.0 - s)
        return carry

    lax.fori_loop(0, NUM_RELATION, _reg_row, 0)

    iota16 = lax.broadcasted_iota(jnp.int32, (16,), 0)
    zeros16 = jnp.zeros((16,), jnp.int32)
    ones16 = jnp.full((16,), 1, jnp.int32)
    osems = (so0, so1)
    ocopies = {}

    for c in range(NCH):
        if c + NBUF - 1 < NCH:
            _fire(c + NBUF - 1)
        cpx, cpy = gcopies[c]
        cpx.wait()
        cpy.wait()
        gb = c % NBUF
        xr, yr = xb.at[gb], yb.at[gb]
        ob = c % 2
        if c - 2 >= 0:
            for cp in ocopies[c - 2]:
                cp.wait()
        for g in range(GPC):
            rvec = r_v[pl.ds((c * GPC + g) * 16, 16)]
            lvec = l_v[pl.ds((c * GPC + g) * 16, 16)]
            rows = g * 16 + iota16

            def _hbody(k, acc, xr=xr, yr=yr, rvec=rvec, rows=rows):
                for u in range(HU):
                    col = (iota16 + (k * HU + u)) & (HIDDEN - 1)
                    vx = plsc.load_gather(xr, [rows, col])
                    vy = plsc.load_gather(yr, [rows, col])
                    vre = plsc.load_gather(wr_v, [rvec, col])
                    acc = acc + vx * vy * vre
                return acc

            acc = lax.fori_loop(0, HIDDEN // HU, _hbody,
                                jnp.zeros((16,), jnp.float32))
            p = 1.0 / (1.0 + jnp.exp(-acc))
            plsc.store_scatter(lg_v.at[ob], [rows, zeros16], p)
            plsc.store_scatter(lg_v.at[ob], [rows, ones16], 1.0 - p)
            p_v[ob, g] = p
            ch_v[ob, g] = jnp.where(lvec == 0, p, 1.0 - p)
        ocopies[c] = (
            pltpu.async_copy(lg_v.at[ob], out_lg.at[pl.ds(base + c * CH, CH)],
                             osems[ob]),
            pltpu.async_copy(p_v.at[ob], out_p.at[pl.ds(obase + c * GPC, GPC)],
                             osems[ob]),
            pltpu.async_copy(ch_v.at[ob], out_ch.at[pl.ds(obase + c * GPC, GPC)],
                             osems[ob]),
        )
    for c in (NCH - 2, NCH - 1):
        for cp in ocopies[c]:
            cp.wait()


def _loss_body(p_ref, ch_ref, loss_ref):
    p = p_ref[...]
    lse = jnp.log(jnp.exp(p) + jnp.exp(1.0 - p))
    loss_ref[...] = (jnp.sum(lse - ch_ref[...]) * (1.0 / BATCH)).reshape(1, 1)


_tc_loss = pl.pallas_call(
    _loss_body,
    out_shape=jax.ShapeDtypeStruct((1, 1), jnp.float32),
)


def kernel(x, y, r, l, Wx, Wr):
    logits, preds, chosen = _sc_fused(
        x.astype(jnp.int32), y.astype(jnp.int32),
        r.astype(jnp.int32), l.astype(jnp.int32), Wx, Wr)
    loss = _tc_loss(preds, chosen)
    return logits, loss.reshape(())


# speedup vs baseline: 6.5787x; 2.2584x over previous
import functools

import jax
import jax.numpy as jnp
from jax import lax
from jax.experimental import pallas as pl
from jax.experimental.pallas import tpu as pltpu
from jax.experimental.pallas import tpu_sc as plsc

NUM_NODE = 10000
NUM_RELATION = 64
HIDDEN = 128
BATCH = 16384

NC, NS = 2, 16
NW = NC * NS
BPW = BATCH // NW
CH = 64
NCH = BPW // CH
GPC = CH // 16
HU = 4
NBUF = 3

_sc_mesh = plsc.VectorSubcoreMesh(core_axis_name="c", subcore_axis_name="s")


@functools.partial(
    pl.kernel,
    mesh=_sc_mesh,
    compiler_params=pltpu.CompilerParams(needs_layout_passes=False),
    out_type=(
        jax.ShapeDtypeStruct((BATCH, 2), jnp.float32),
        jax.ShapeDtypeStruct((BATCH // 16, 16), jnp.float32),
        jax.ShapeDtypeStruct((BATCH // 16, 16), jnp.float32),
    ),
    scratch_types=[
        pltpu.VMEM((2, NCH * CH), jnp.int32),
        pltpu.VMEM((BPW,), jnp.int32),
        pltpu.VMEM((BPW,), jnp.int32),
        pltpu.VMEM((NUM_RELATION, HIDDEN), jnp.float32),
        pltpu.VMEM((NBUF, CH, HIDDEN), jnp.float32),
        pltpu.VMEM((NBUF, CH, HIDDEN), jnp.float32),
        pltpu.VMEM((2, CH, 2), jnp.float32),
        pltpu.VMEM((2, GPC, 16), jnp.float32),
        pltpu.VMEM((2, GPC, 16), jnp.float32),
        pltpu.SemaphoreType.DMA,
        pltpu.SemaphoreType.DMA,
        pltpu.SemaphoreType.DMA,
        pltpu.SemaphoreType.DMA,
        pltpu.SemaphoreType.DMA,
        pltpu.SemaphoreType.DMA,
    ],
)
def _sc_fused(x_hbm, y_hbm, r_hbm, l_hbm, wx_hbm, wr_hbm,
              out_lg, out_p, out_ch,
              idx_v, r_v, l_v, wr_v, xb, yb, lg_v, p_v, ch_v,
              s0, s1, s2, so0, so1, si):
    wid = lax.axis_index("s") * NC + lax.axis_index("c")
    base = wid * BPW
    obase = wid * (BPW // 16)

    st = [
        pltpu.async_copy(x_hbm.at[pl.ds(base, BPW)], idx_v.at[0], si),
        pltpu.async_copy(y_hbm.at[pl.ds(base, BPW)], idx_v.at[1], si),
        pltpu.async_copy(r_hbm.at[pl.ds(base, BPW)], r_v, si),
        pltpu.async_copy(l_hbm.at[pl.ds(base, BPW)], l_v, si),
        pltpu.async_copy(wr_hbm, wr_v, si),
    ]
    for cp in st[:2]:
        cp.wait()

    gsems = (s0, s1, s2)
    gcopies = {}

    def _fire(c):
        b = c % NBUF
        cpx = pltpu.async_copy(
            wx_hbm.at[idx_v.at[0, pl.ds(c * CH, CH)]], xb.at[b], gsems[b])
        cpy = pltpu.async_copy(
            wx_hbm.at[idx_v.at[1, pl.ds(c * CH, CH)]], yb.at[b], gsems[b])
        gcopies[c] = (cpx, cpy)

    for c in range(NBUF - 1):
        _fire(c)
    for cp in st[2:]:
        cp.wait()

    def _reg_row(k, carry):
        for j in range(HIDDEN // 16):
            v = wr_v[k, pl.ds(j * 16, 16)]
            v = jnp.clip(v, -6.0, 6.0)
            s = 1.0 / (1.0 + jnp.exp(-v))
            wr_v[k, pl.ds(j * 16, 16)] = s * (1.0 - s)
        return carry

    lax.fori_loop(0, NUM_RELATION, _reg_row, 0)

    iota16 = lax.broadcasted_iota(jnp.int32, (16,), 0)
    zeros16 = jnp.zeros((16,), jnp.int32)
    ones16 = jnp.full((16,), 1, jnp.int32)
    osems = (so0, so1)
    ocopies = {}

    for c in range(NCH):
        if c + NBUF - 1 < NCH:
            _fire(c + NBUF - 1)
        cpx, cpy = gcopies[c]
        cpx.wait()
        cpy.wait()
        gb = c % NBUF
        xr, yr = xb.at[gb], yb.at[gb]
        ob = c % 2
        if c - 2 >= 0:
            for cp in ocopies[c - 2]:
                cp.wait()
        for g in range(GPC):
            rvec = r_v[pl.ds((c * GPC + g) * 16, 16)]
            lvec = l_v[pl.ds((c * GPC + g) * 16, 16)]
            rows = g * 16 + iota16

            def _hbody(k, accs, xr=xr, yr=yr, rvec=rvec, rows=rows):
                out = []
                for u in range(HU):
                    col = (iota16 + (k * HU + u)) & (HIDDEN - 1)
                    vx = plsc.load_gather(xr, [rows, col])
                    vy = plsc.load_gather(yr, [rows, col])
                    vre = plsc.load_gather(wr_v, [rvec, col])
                    out.append(accs[u] + vx * vy * vre)
                return tuple(out)

            accs = lax.fori_loop(
                0, HIDDEN // HU, _hbody,
                tuple(jnp.zeros((16,), jnp.float32) for _ in range(HU)))
            acc = (accs[0] + accs[1]) + (accs[2] + accs[3])
            p = 1.0 / (1.0 + jnp.exp(-acc))
            plsc.store_scatter(lg_v.at[ob], [rows, zeros16], p)
            plsc.store_scatter(lg_v.at[ob], [rows, ones16], 1.0 - p)
            p_v[ob, g] = p
            ch_v[ob, g] = jnp.where(lvec == 0, p, 1.0 - p)
        ocopies[c] = (
            pltpu.async_copy(lg_v.at[ob], out_lg.at[pl.ds(base + c * CH, CH)],
                             osems[ob]),
            pltpu.async_copy(p_v.at[ob], out_p.at[pl.ds(obase + c * GPC, GPC)],
                             osems[ob]),
            pltpu.async_copy(ch_v.at[ob], out_ch.at[pl.ds(obase + c * GPC, GPC)],
                             osems[ob]),
        )
    for c in (NCH - 2, NCH - 1):
        for cp in ocopies[c]:
            cp.wait()


def _loss_body(p_ref, ch_ref, loss_ref):
    p = p_ref[...]
    lse = jnp.log(jnp.exp(p) + jnp.exp(1.0 - p))
    loss_ref[...] = (jnp.sum(lse - ch_ref[...]) * (1.0 / BATCH)).reshape(1, 1)


_tc_loss = pl.pallas_call(
    _loss_body,
    out_shape=jax.ShapeDtypeStruct((1, 1), jnp.float32),
)


def kernel(x, y, r, l, Wx, Wr):
    logits, preds, chosen = _sc_fused(
        x.astype(jnp.int32), y.astype(jnp.int32),
        r.astype(jnp.int32), l.astype(jnp.int32), Wx, Wr)
    loss = _tc_loss(preds, chosen)
    return logits, loss.reshape(())
